# baseline (speedup 1.0000x reference)
import jax
import jax.numpy as jnp
from jax.experimental import pallas as pl

_BS = 256


def _add_body(x_ref, pe_ref, o_ref):
    i = pl.program_id(0)
    o_ref[...] = x_ref[...] + pe_ref[:, pl.ds(i * _BS, _BS), :]


def kernel(inputs, pos_embed):
    B, S, D = inputs.shape
    grid = (S // _BS,)
    return pl.pallas_call(
        _add_body,
        grid=grid,
        in_specs=[
            pl.BlockSpec((B, _BS, D), lambda i: (0, i, 0)),
            pl.BlockSpec((1, S, D), lambda i: (0, 0, 0)),
        ],
        out_specs=pl.BlockSpec((B, _BS, D), lambda i: (0, i, 0)),
        out_shape=jax.ShapeDtypeStruct((B, S, D), inputs.dtype),
    )(inputs, pos_embed[None])
